# baseline (speedup 1.0000x reference)
import functools

import jax
import jax.numpy as jnp
from jax import lax
from jax.experimental import pallas as pl
from jax.experimental.pallas import tpu as pltpu
from jax.experimental.pallas import tpu_sc as plsc

N_NODE = 10000
NPAD = 10112
E = 320000
H = 128
CHUNK = 128
NCHUNK = E // CHUNK
NSUB = 16
CPS = 162
NCHUNK_PAD = CPS * NSUB
E_PAD = NCHUNK_PAD * CHUNK
SEG = NPAD // NSUB
DEG_PAD = 10240
SEG_DEG = DEG_PAD // NSUB

_MESH = plsc.VectorSubcoreMesh(core_axis_name="c", subcore_axis_name="s")


_NB_DEG = 6


@functools.partial(
    pl.kernel,
    out_type=[jax.ShapeDtypeStruct((DEG_PAD,), jnp.float32)] * 4,
    mesh=_MESH,
    scratch_types=[
        pltpu.VMEM_SHARED((DEG_PAD,), jnp.float32),
        pltpu.VMEM_SHARED((DEG_PAD,), jnp.float32),
        [pltpu.VMEM((CHUNK,), jnp.int32)] * _NB_DEG,
        [pltpu.VMEM((CHUNK,), jnp.int32)] * _NB_DEG,
        pltpu.VMEM((CHUNK,), jnp.float32),
        [pltpu.SemaphoreType.DMA] * _NB_DEG,
        [pltpu.SemaphoreType.DMA] * _NB_DEG,
        [pltpu.SemaphoreType.DMA] * _NB_DEG,
        [pltpu.SemaphoreType.DMA] * _NB_DEG,
    ],
)
def _sc_degrees(rs, rd, qs, qd, zeros1, out0, out1, out2, out3,
                dega, degb, idxa, idxb, ones_v, ia, ib, sa, sb):
    cid = lax.axis_index("c")
    sid = lax.axis_index("s")
    base = sid * CPS
    for i in range(CHUNK // 16):
        ones_v[pl.ds(i * 16, 16)] = jnp.full((16,), 1.0, jnp.float32)
    pltpu.sync_copy(zeros1.at[pl.ds(sid * SEG_DEG, SEG_DEG)], dega.at[pl.ds(sid * SEG_DEG, SEG_DEG)])
    pltpu.sync_copy(zeros1.at[pl.ds(sid * SEG_DEG, SEG_DEG)], degb.at[pl.ds(sid * SEG_DEG, SEG_DEG)])
    plsc.subcore_barrier()

    def run(s2d, d2d):
        for b in range(_NB_DEG):
            pltpu.async_copy(s2d.at[base + b], idxa[b], ia[b])
            pltpu.async_copy(d2d.at[base + b], idxb[b], ib[b])

        def body(i, _):
            descs = []
            for b in range(_NB_DEG):
                j = i * _NB_DEG + b
                c = base + j
                pltpu.make_async_copy(s2d.at[c], idxa[b], ia[b]).wait()
                pltpu.make_async_copy(d2d.at[c], idxb[b], ib[b]).wait()
                descs.append(
                    (pltpu.async_copy(ones_v, dega.at[idxa[b]], sa[b], add=True),
                     pltpu.async_copy(ones_v, degb.at[idxb[b]], sb[b], add=True)))
            for b in range(_NB_DEG):
                j = i * _NB_DEG + b
                c = base + j
                descs[b][0].wait()
                descs[b][1].wait()

                @pl.when(j + _NB_DEG < CPS)
                def _():
                    pltpu.async_copy(s2d.at[c + _NB_DEG], idxa[b], ia[b])
                    pltpu.async_copy(d2d.at[c + _NB_DEG], idxb[b], ib[b])
            return 0
        lax.fori_loop(0, CPS // _NB_DEG, body, 0)

    @pl.when(cid == 0)
    def _():
        run(rs, rd)

    @pl.when(cid == 1)
    def _():
        run(qs, qd)

    plsc.subcore_barrier()

    @pl.when(cid == 0)
    def _():
        pltpu.sync_copy(dega.at[pl.ds(sid * SEG_DEG, SEG_DEG)], out0.at[pl.ds(sid * SEG_DEG, SEG_DEG)])
        pltpu.sync_copy(degb.at[pl.ds(sid * SEG_DEG, SEG_DEG)], out1.at[pl.ds(sid * SEG_DEG, SEG_DEG)])

    @pl.when(cid == 1)
    def _():
        pltpu.sync_copy(dega.at[pl.ds(sid * SEG_DEG, SEG_DEG)], out2.at[pl.ds(sid * SEG_DEG, SEG_DEG)])
        pltpu.sync_copy(degb.at[pl.ds(sid * SEG_DEG, SEG_DEG)], out3.at[pl.ds(sid * SEG_DEG, SEG_DEG)])


_NB_AGG = 3


@functools.partial(
    pl.kernel,
    out_type=[jax.ShapeDtypeStruct((NPAD, H), jnp.float32)] * 2,
    mesh=_MESH,
    scratch_types=[
        pltpu.VMEM_SHARED((NPAD, H), jnp.float32),
        [pltpu.VMEM((CHUNK,), jnp.int32)] * _NB_AGG,
        [pltpu.VMEM((CHUNK,), jnp.int32)] * _NB_AGG,
        [pltpu.VMEM((CHUNK, H), jnp.float32)] * _NB_AGG,
        [pltpu.SemaphoreType.DMA] * _NB_AGG,
        [pltpu.SemaphoreType.DMA] * _NB_AGG,
        [pltpu.SemaphoreType.DMA] * _NB_AGG,
        [pltpu.SemaphoreType.DMA] * _NB_AGG,
    ],
)
def _sc_aggregate(hs_user, hs_item, rs, rd, qs, qd, zeros2,
                  agg_item, agg_user, accum, idxs, idxd, rows, ia, ib, gs, ss):
    cid = lax.axis_index("c")
    sid = lax.axis_index("s")
    base = sid * CPS
    pltpu.sync_copy(zeros2.at[pl.ds(sid * SEG, SEG)],
                    accum.at[pl.ds(sid * SEG, SEG)])
    plsc.subcore_barrier()

    def run(tbl, s2d, d2d):
        for b in range(_NB_AGG):
            pltpu.async_copy(s2d.at[base + b], idxs[b], ia[b])
            pltpu.async_copy(d2d.at[base + b], idxd[b], ib[b])

        def body(i, _):
            gdescs = []
            for b in range(_NB_AGG):
                c = base + i * _NB_AGG + b
                pltpu.make_async_copy(s2d.at[c], idxs[b], ia[b]).wait()
                gdescs.append(pltpu.async_copy(tbl.at[idxs[b]], rows[b], gs[b]))
            sdescs = []
            for b in range(_NB_AGG):
                c = base + i * _NB_AGG + b
                gdescs[b].wait()
                pltpu.make_async_copy(d2d.at[c], idxd[b], ib[b]).wait()
                sdescs.append(pltpu.async_copy(rows[b], accum.at[idxd[b]],
                                               ss[b], add=True))
            for b in range(_NB_AGG):
                j = i * _NB_AGG + b
                c = base + j
                sdescs[b].wait()

                @pl.when(j + _NB_AGG < CPS)
                def _():
                    pltpu.async_copy(s2d.at[c + _NB_AGG], idxs[b], ia[b])
                    pltpu.async_copy(d2d.at[c + _NB_AGG], idxd[b], ib[b])
            return 0
        lax.fori_loop(0, CPS // _NB_AGG, body, 0)

    @pl.when(cid == 0)
    def _():
        run(hs_user, rs, rd)

    @pl.when(cid == 1)
    def _():
        run(hs_item, qs, qd)

    plsc.subcore_barrier()

    @pl.when(cid == 0)
    def _():
        pltpu.sync_copy(accum.at[pl.ds(sid * SEG, SEG)],
                        agg_item.at[pl.ds(sid * SEG, SEG)])

    @pl.when(cid == 1)
    def _():
        pltpu.sync_copy(accum.at[pl.ds(sid * SEG, SEG)],
                        agg_user.at[pl.ds(sid * SEG, SEG)])


_NB_EDGE = 3
_CPS_EDGE = NCHUNK_PAD // 32


@functools.partial(
    pl.kernel,
    out_type=jax.ShapeDtypeStruct((E_PAD, H), jnp.float32),
    mesh=_MESH,
    scratch_types=[
        [pltpu.VMEM((CHUNK,), jnp.int32)] * _NB_EDGE,
        [pltpu.VMEM((CHUNK,), jnp.int32)] * _NB_EDGE,
        [pltpu.VMEM((CHUNK, H), jnp.float32)] * _NB_EDGE,
        [pltpu.SemaphoreType.DMA] * _NB_EDGE,
        [pltpu.SemaphoreType.DMA] * _NB_EDGE,
        [pltpu.SemaphoreType.DMA] * _NB_EDGE,
        [pltpu.SemaphoreType.DMA] * _NB_EDGE,
        [pltpu.SemaphoreType.DMA] * _NB_EDGE,
    ],
)
def _sc_edge_gather(u_tbl, v_tbl, s2d, d2d, out,
                    idxs, idxd, rows, ia, ib, us, vs, os):
    cid = lax.axis_index("c")
    sid = lax.axis_index("s")
    base = (cid * NSUB + sid) * _CPS_EDGE

    for b in range(_NB_EDGE):
        pltpu.async_copy(s2d.at[base + b], idxs[b], ia[b])
        pltpu.async_copy(d2d.at[base + b], idxd[b], ib[b])

    def body(i, _):
        udescs = []
        for b in range(_NB_EDGE):
            c = base + i * _NB_EDGE + b
            pltpu.make_async_copy(s2d.at[c], idxs[b], ia[b]).wait()
            udescs.append(pltpu.async_copy(u_tbl.at[idxs[b]], rows[b], us[b]))
        vdescs = []
        for b in range(_NB_EDGE):
            c = base + i * _NB_EDGE + b
            udescs[b].wait()
            pltpu.make_async_copy(d2d.at[c], idxd[b], ib[b]).wait()
            vdescs.append(pltpu.async_copy(v_tbl.at[idxd[b]], rows[b],
                                           vs[b], add=True))
        odescs = []
        for b in range(_NB_EDGE):
            c = base + i * _NB_EDGE + b
            vdescs[b].wait()
            odescs.append(pltpu.async_copy(
                rows[b], out.at[pl.ds(c * CHUNK, CHUNK)], os[b]))
        for b in range(_NB_EDGE):
            j = i * _NB_EDGE + b
            c = base + j
            odescs[b].wait()

            @pl.when(j + _NB_EDGE < _CPS_EDGE)
            def _():
                pltpu.async_copy(s2d.at[c + _NB_EDGE], idxs[b], ia[b])
                pltpu.async_copy(d2d.at[c + _NB_EDGE], idxd[b], ib[b])
        return 0
    lax.fori_loop(0, _CPS_EDGE // _NB_EDGE, body, 0)


_ROW_BLK = 1264
_ROW_GRID = NPAD // _ROW_BLK
_row_spec = pl.BlockSpec((_ROW_BLK, H), lambda i: (i, 0))
_deg_spec = pl.BlockSpec((_ROW_BLK, 1), lambda i: (i, 0))
_w_spec = pl.BlockSpec((H, H), lambda i: (0, 0))
_b_spec = pl.BlockSpec((1, H), lambda i: (0, 0))


def _tc_proj_body(xu, xi, wu, bu, wi, bi, deg_ru, deg_qi, hsu, hsi):
    hu = jnp.dot(xu[...], wu[...], preferred_element_type=jnp.float32) + bu[...]
    hsu[...] = hu * deg_ru[...]
    hi = jnp.dot(xi[...], wi[...], preferred_element_type=jnp.float32) + bi[...]
    hsi[...] = hi * deg_qi[...]


def _tc_proj(xu, xi, wu, bu, wi, bi, deg_ru, deg_qi):
    return pl.pallas_call(
        _tc_proj_body,
        grid=(_ROW_GRID,),
        in_specs=[_row_spec, _row_spec, _w_spec, _b_spec, _w_spec, _b_spec,
                  _deg_spec, _deg_spec],
        out_specs=[_row_spec] * 2,
        out_shape=[jax.ShapeDtypeStruct((NPAD, H), jnp.float32)] * 2,
    )(xu, xi, wu, bu.reshape(1, H), wi, bi.reshape(1, H), deg_ru, deg_qi)


def _tc_layer_body(relu, aggi, aggu, w_rates, b_rates, w_rated, b_rated,
                   deg_ri, deg_qu, deg_qi, deg_ru, hsu, hsi):
    hi = (jnp.dot(aggi[...], w_rates[...], preferred_element_type=jnp.float32)
          * deg_ri[...] + b_rates[...])
    hu = (jnp.dot(aggu[...], w_rated[...], preferred_element_type=jnp.float32)
          * deg_qu[...] + b_rated[...])
    if relu:
        hi = jnp.maximum(hi, 0.0)
        hu = jnp.maximum(hu, 0.0)
    hsi[...] = hi * deg_qi[...]
    hsu[...] = hu * deg_ru[...]


def _tc_layer(aggi, aggu, w_rates, b_rates, w_rated, b_rated,
              deg_ri, deg_qu, deg_qi, deg_ru, relu):
    return pl.pallas_call(
        functools.partial(_tc_layer_body, relu),
        grid=(_ROW_GRID,),
        in_specs=[_row_spec, _row_spec, _w_spec, _b_spec, _w_spec, _b_spec,
                  _deg_spec, _deg_spec, _deg_spec, _deg_spec],
        out_specs=[_row_spec] * 2,
        out_shape=[jax.ShapeDtypeStruct((NPAD, H), jnp.float32)] * 2,
    )(aggi, aggu, w_rates, b_rates.reshape(1, H), w_rated, b_rated.reshape(1, H),
      deg_ri, deg_qu, deg_qi, deg_ru)


def _tc_final_body(aggi, aggu, w_rates, b_rates, w_rated, b_rated,
                   deg_ri, deg_qu, ftw_u, ftw_i, ft_b, u_out, v_out):
    hi = (jnp.dot(aggi[...], w_rates[...], preferred_element_type=jnp.float32)
          * deg_ri[...] + b_rates[...])
    hu = (jnp.dot(aggu[...], w_rated[...], preferred_element_type=jnp.float32)
          * deg_qu[...] + b_rated[...])
    u_out[...] = jnp.dot(hu, ftw_u[...], preferred_element_type=jnp.float32)
    v_out[...] = jnp.dot(hi, ftw_i[...], preferred_element_type=jnp.float32) + ft_b[...]


def _tc_final(aggi, aggu, w_rates, b_rates, w_rated, b_rated,
              deg_ri, deg_qu, ftw_u, ftw_i, ft_b):
    return pl.pallas_call(
        _tc_final_body,
        grid=(_ROW_GRID,),
        in_specs=[_row_spec, _row_spec, _w_spec, _b_spec, _w_spec, _b_spec,
                  _deg_spec, _deg_spec, _w_spec, _w_spec, _b_spec],
        out_specs=[_row_spec] * 2,
        out_shape=[jax.ShapeDtypeStruct((NPAD, H), jnp.float32)] * 2,
    )(aggi, aggu, w_rates, b_rates.reshape(1, H), w_rated, b_rated.reshape(1, H),
      deg_ri, deg_qu, ftw_u, ftw_i, ft_b.reshape(1, H))


_SCORE_BLK = 3200
_SCORE_GRID = E // _SCORE_BLK


def _tc_score_body(g, bn_g, bn_b, bn_mean, bn_var, emb, w1, b1, w2, b2,
                   s_out):
    t = jnp.maximum(g[...], 0.0)
    t = (t - bn_mean[...]) / jnp.sqrt(bn_var[...] + 1e-5) * bn_g[...] + bn_b[...]
    c = t + emb[...]
    z = jnp.maximum(jnp.dot(c, w1[...], preferred_element_type=jnp.float32)
                    + b1[...], 0.0)
    s = jnp.dot(z, w2[...], preferred_element_type=jnp.float32)[:, 0] + b2[0, 0]
    i = pl.program_id(0)
    nrow = _SCORE_BLK // CHUNK
    s_out[pl.ds(i * nrow, nrow), :] = s.reshape(nrow, CHUNK)


def _tc_score(g, bn_g, bn_b, bn_mean, bn_var, emb, w1, b1, w2, b2):
    full = lambda r, c: pl.BlockSpec((r, c), lambda i: (0, 0))
    return pl.pallas_call(
        _tc_score_body,
        grid=(_SCORE_GRID,),
        in_specs=[
            pl.BlockSpec((_SCORE_BLK, H), lambda i: (i, 0)),
            full(1, H), full(1, H), full(1, H), full(1, H), full(1, H),
            full(H, H // 2), full(1, H // 2), full(H // 2, 1), full(1, 1),
        ],
        out_specs=pl.BlockSpec((NCHUNK, CHUNK), lambda i: (0, 0)),
        out_shape=jax.ShapeDtypeStruct((NCHUNK, CHUNK), jnp.float32),
    )(g, bn_g.reshape(1, H), bn_b.reshape(1, H), bn_mean.reshape(1, H),
      bn_var.reshape(1, H), emb.reshape(1, H), w1, b1.reshape(1, H // 2),
      w2, b2.reshape(1, 1))


def kernel(x_user, x_item, rates_src, rates_dst, rated_src, rated_dst,
           pos_src, pos_dst, neg_src, neg_dst,
           ip_user_W, ip_user_b, ip_item_W, ip_item_b,
           l0_rates_W, l0_rates_b, l0_rated_W, l0_rated_b,
           l1_rates_W, l1_rates_b, l1_rated_W, l1_rated_b,
           l2_rates_W, l2_rates_b, l2_rated_W, l2_rated_b,
           ft_W, ft_b, bn_g, bn_b, bn_mean, bn_var, emb,
           ep_W1, ep_b1, ep_W2, ep_b2):
    fill = N_NODE + (jnp.arange(E_PAD - E, dtype=jnp.int32) % (NPAD - N_NODE))
    c2 = lambda a: jnp.concatenate(
        [a.astype(jnp.int32), fill]).reshape(NCHUNK_PAD, CHUNK)
    rs, rd = c2(rates_src), c2(rates_dst)
    qs, qd = c2(rated_src), c2(rated_dst)
    ps, pd = c2(pos_src), c2(pos_dst)
    ns, nd = c2(neg_src), c2(neg_dst)
    zeros1 = jnp.zeros((DEG_PAD,), jnp.float32)
    zeros2 = jnp.zeros((NPAD, H), jnp.float32)

    deg_ru, deg_ri, deg_qi, deg_qu = _sc_degrees(rs, rd, qs, qd, zeros1)
    col = lambda a: (jnp.clip(a, 1.0, None) ** -0.5)[:NPAD].reshape(NPAD, 1)
    deg_ru, deg_ri, deg_qi, deg_qu = col(deg_ru), col(deg_ri), col(deg_qi), col(deg_qu)

    xpad = jnp.zeros((NPAD - N_NODE, H), jnp.float32)
    xu_p = jnp.concatenate([x_user, xpad])
    xi_p = jnp.concatenate([x_item, xpad])
    hsu, hsi = _tc_proj(xu_p, xi_p, ip_user_W, ip_user_b,
                        ip_item_W, ip_item_b, deg_ru, deg_qi)

    layer_w = [(l0_rates_W, l0_rates_b, l0_rated_W, l0_rated_b),
               (l1_rates_W, l1_rates_b, l1_rated_W, l1_rated_b)]
    for w_rates, b_rates, w_rated, b_rated in layer_w:
        aggi, aggu = _sc_aggregate(hsu, hsi, rs, rd, qs, qd, zeros2)
        hsu, hsi = _tc_layer(aggi, aggu, w_rates, b_rates, w_rated, b_rated,
                             deg_ri, deg_qu, deg_qi, deg_ru, relu=True)

    aggi, aggu = _sc_aggregate(hsu, hsi, rs, rd, qs, qd, zeros2)
    u_tbl, v_tbl = _tc_final(aggi, aggu, l2_rates_W, l2_rates_b,
                             l2_rated_W, l2_rated_b, deg_ri, deg_qu,
                             ft_W[:H], ft_W[H:], ft_b)

    g_pos = _sc_edge_gather(u_tbl, v_tbl, ps, pd)
    g_neg = _sc_edge_gather(u_tbl, v_tbl, ns, nd)
    pos2 = _tc_score(g_pos, bn_g, bn_b, bn_mean, bn_var, emb,
                     ep_W1, ep_b1, ep_W2, ep_b2)
    neg2 = _tc_score(g_neg, bn_g, bn_b, bn_mean, bn_var, emb,
                     ep_W1, ep_b1, ep_W2, ep_b2)
    return pos2.reshape(E), neg2.reshape(E)


# speedup vs baseline: 1.0130x; 1.0130x over previous
import functools

import jax
import jax.numpy as jnp
from jax import lax
from jax.experimental import pallas as pl
from jax.experimental.pallas import tpu as pltpu
from jax.experimental.pallas import tpu_sc as plsc

N_NODE = 10000
NPAD = 10112
E = 320000
H = 128
CHUNK = 128
NCHUNK = E // CHUNK
NSUB = 16
CPS = 162
NCHUNK_PAD = CPS * NSUB
E_PAD = NCHUNK_PAD * CHUNK
SEG = NPAD // NSUB
DEG_PAD = 10240
SEG_DEG = DEG_PAD // NSUB

_MESH = plsc.VectorSubcoreMesh(core_axis_name="c", subcore_axis_name="s")


_NB_DEG = 6


@functools.partial(
    pl.kernel,
    out_type=[jax.ShapeDtypeStruct((DEG_PAD,), jnp.float32)] * 4,
    mesh=_MESH,
    scratch_types=[
        pltpu.VMEM_SHARED((DEG_PAD,), jnp.float32),
        pltpu.VMEM_SHARED((DEG_PAD,), jnp.float32),
        [pltpu.VMEM((CHUNK,), jnp.int32)] * _NB_DEG,
        [pltpu.VMEM((CHUNK,), jnp.int32)] * _NB_DEG,
        pltpu.VMEM((CHUNK,), jnp.float32),
        [pltpu.SemaphoreType.DMA] * _NB_DEG,
        [pltpu.SemaphoreType.DMA] * _NB_DEG,
        [pltpu.SemaphoreType.DMA] * _NB_DEG,
        [pltpu.SemaphoreType.DMA] * _NB_DEG,
    ],
)
def _sc_degrees(rs, rd, qs, qd, zeros1, out0, out1, out2, out3,
                dega, degb, idxa, idxb, ones_v, ia, ib, sa, sb):
    cid = lax.axis_index("c")
    sid = lax.axis_index("s")
    base = sid * CPS
    for i in range(CHUNK // 16):
        ones_v[pl.ds(i * 16, 16)] = jnp.full((16,), 1.0, jnp.float32)
    pltpu.sync_copy(zeros1.at[pl.ds(sid * SEG_DEG, SEG_DEG)], dega.at[pl.ds(sid * SEG_DEG, SEG_DEG)])
    pltpu.sync_copy(zeros1.at[pl.ds(sid * SEG_DEG, SEG_DEG)], degb.at[pl.ds(sid * SEG_DEG, SEG_DEG)])
    plsc.subcore_barrier()

    def run(s2d, d2d):
        for b in range(_NB_DEG):
            pltpu.async_copy(s2d.at[base + b], idxa[b], ia[b])
            pltpu.async_copy(d2d.at[base + b], idxb[b], ib[b])

        def body(i, _):
            descs = []
            for b in range(_NB_DEG):
                j = i * _NB_DEG + b
                c = base + j
                pltpu.make_async_copy(s2d.at[c], idxa[b], ia[b]).wait()
                pltpu.make_async_copy(d2d.at[c], idxb[b], ib[b]).wait()
                descs.append(
                    (pltpu.async_copy(ones_v, dega.at[idxa[b]], sa[b], add=True),
                     pltpu.async_copy(ones_v, degb.at[idxb[b]], sb[b], add=True)))
            for b in range(_NB_DEG):
                j = i * _NB_DEG + b
                c = base + j
                descs[b][0].wait()
                descs[b][1].wait()

                @pl.when(j + _NB_DEG < CPS)
                def _():
                    pltpu.async_copy(s2d.at[c + _NB_DEG], idxa[b], ia[b])
                    pltpu.async_copy(d2d.at[c + _NB_DEG], idxb[b], ib[b])
            return 0
        lax.fori_loop(0, CPS // _NB_DEG, body, 0)

    @pl.when(cid == 0)
    def _():
        run(rs, rd)

    @pl.when(cid == 1)
    def _():
        run(qs, qd)

    plsc.subcore_barrier()

    @pl.when(cid == 0)
    def _():
        pltpu.sync_copy(dega.at[pl.ds(sid * SEG_DEG, SEG_DEG)], out0.at[pl.ds(sid * SEG_DEG, SEG_DEG)])
        pltpu.sync_copy(degb.at[pl.ds(sid * SEG_DEG, SEG_DEG)], out1.at[pl.ds(sid * SEG_DEG, SEG_DEG)])

    @pl.when(cid == 1)
    def _():
        pltpu.sync_copy(dega.at[pl.ds(sid * SEG_DEG, SEG_DEG)], out2.at[pl.ds(sid * SEG_DEG, SEG_DEG)])
        pltpu.sync_copy(degb.at[pl.ds(sid * SEG_DEG, SEG_DEG)], out3.at[pl.ds(sid * SEG_DEG, SEG_DEG)])


_NB_AGG = 3


@functools.partial(
    pl.kernel,
    out_type=[jax.ShapeDtypeStruct((NPAD, H), jnp.float32)] * 2,
    mesh=_MESH,
    scratch_types=[
        pltpu.VMEM_SHARED((NPAD, H), jnp.float32),
        [pltpu.VMEM((CHUNK,), jnp.int32)] * _NB_AGG,
        [pltpu.VMEM((CHUNK,), jnp.int32)] * _NB_AGG,
        [pltpu.VMEM((CHUNK, H), jnp.float32)] * _NB_AGG,
        [pltpu.SemaphoreType.DMA] * _NB_AGG,
        [pltpu.SemaphoreType.DMA] * _NB_AGG,
        [pltpu.SemaphoreType.DMA] * _NB_AGG,
        [pltpu.SemaphoreType.DMA] * _NB_AGG,
    ],
)
def _sc_aggregate(hs_user, hs_item, rs, rd, qs, qd, zeros2,
                  agg_item, agg_user, accum, idxs, idxd, rows, ia, ib, gs, ss):
    cid = lax.axis_index("c")
    sid = lax.axis_index("s")
    base = sid * CPS
    pltpu.sync_copy(zeros2.at[pl.ds(sid * SEG, SEG)],
                    accum.at[pl.ds(sid * SEG, SEG)])
    plsc.subcore_barrier()

    def run(tbl, s2d, d2d):
        for b in range(_NB_AGG):
            pltpu.async_copy(s2d.at[base + b], idxs[b], ia[b])
            pltpu.async_copy(d2d.at[base + b], idxd[b], ib[b])

        def body(i, _):
            gdescs = []
            for b in range(_NB_AGG):
                c = base + i * _NB_AGG + b
                pltpu.make_async_copy(s2d.at[c], idxs[b], ia[b]).wait()
                gdescs.append(pltpu.async_copy(tbl.at[idxs[b]], rows[b], gs[b]))
            sdescs = []
            for b in range(_NB_AGG):
                c = base + i * _NB_AGG + b
                gdescs[b].wait()
                pltpu.make_async_copy(d2d.at[c], idxd[b], ib[b]).wait()
                sdescs.append(pltpu.async_copy(rows[b], accum.at[idxd[b]],
                                               ss[b], add=True))
            for b in range(_NB_AGG):
                j = i * _NB_AGG + b
                c = base + j
                sdescs[b].wait()

                @pl.when(j + _NB_AGG < CPS)
                def _():
                    pltpu.async_copy(s2d.at[c + _NB_AGG], idxs[b], ia[b])
                    pltpu.async_copy(d2d.at[c + _NB_AGG], idxd[b], ib[b])
            return 0
        lax.fori_loop(0, CPS // _NB_AGG, body, 0)

    @pl.when(cid == 0)
    def _():
        run(hs_user, rs, rd)

    @pl.when(cid == 1)
    def _():
        run(hs_item, qs, qd)

    plsc.subcore_barrier()

    @pl.when(cid == 0)
    def _():
        pltpu.sync_copy(accum.at[pl.ds(sid * SEG, SEG)],
                        agg_item.at[pl.ds(sid * SEG, SEG)])

    @pl.when(cid == 1)
    def _():
        pltpu.sync_copy(accum.at[pl.ds(sid * SEG, SEG)],
                        agg_user.at[pl.ds(sid * SEG, SEG)])


_NB_EDGE = 6


@functools.partial(
    pl.kernel,
    out_type=[jax.ShapeDtypeStruct((E_PAD, H), jnp.float32)] * 2,
    mesh=_MESH,
    scratch_types=[
        [pltpu.VMEM((CHUNK,), jnp.int32)] * _NB_EDGE,
        [pltpu.VMEM((CHUNK,), jnp.int32)] * _NB_EDGE,
        [pltpu.VMEM((CHUNK, H), jnp.float32)] * _NB_EDGE,
        [pltpu.SemaphoreType.DMA] * _NB_EDGE,
        [pltpu.SemaphoreType.DMA] * _NB_EDGE,
        [pltpu.SemaphoreType.DMA] * _NB_EDGE,
        [pltpu.SemaphoreType.DMA] * _NB_EDGE,
        [pltpu.SemaphoreType.DMA] * _NB_EDGE,
    ],
)
def _sc_edge_gather(u_tbl, v_tbl, ps, pd, ns, nd, g_pos, g_neg,
                    idxs, idxd, rows, ia, ib, us, vs, os):
    cid = lax.axis_index("c")
    sid = lax.axis_index("s")
    base = sid * CPS

    def run(s2d, d2d, out):
        for b in range(_NB_EDGE):
            pltpu.async_copy(s2d.at[base + b], idxs[b], ia[b])
            pltpu.async_copy(d2d.at[base + b], idxd[b], ib[b])

        def body(i, _):
            udescs = []
            for b in range(_NB_EDGE):
                c = base + i * _NB_EDGE + b
                pltpu.make_async_copy(s2d.at[c], idxs[b], ia[b]).wait()
                udescs.append(pltpu.async_copy(u_tbl.at[idxs[b]], rows[b], us[b]))
            vdescs = []
            for b in range(_NB_EDGE):
                c = base + i * _NB_EDGE + b
                udescs[b].wait()
                pltpu.make_async_copy(d2d.at[c], idxd[b], ib[b]).wait()
                vdescs.append(pltpu.async_copy(v_tbl.at[idxd[b]], rows[b],
                                               vs[b], add=True))
            odescs = []
            for b in range(_NB_EDGE):
                c = base + i * _NB_EDGE + b
                vdescs[b].wait()
                odescs.append(pltpu.async_copy(
                    rows[b], out.at[pl.ds(c * CHUNK, CHUNK)], os[b]))
            for b in range(_NB_EDGE):
                j = i * _NB_EDGE + b
                c = base + j
                odescs[b].wait()

                @pl.when(j + _NB_EDGE < CPS)
                def _():
                    pltpu.async_copy(s2d.at[c + _NB_EDGE], idxs[b], ia[b])
                    pltpu.async_copy(d2d.at[c + _NB_EDGE], idxd[b], ib[b])
            return 0
        lax.fori_loop(0, CPS // _NB_EDGE, body, 0)

    @pl.when(cid == 0)
    def _():
        run(ps, pd, g_pos)

    @pl.when(cid == 1)
    def _():
        run(ns, nd, g_neg)


_ROW_BLK = 1264
_ROW_GRID = NPAD // _ROW_BLK
_row_spec = pl.BlockSpec((_ROW_BLK, H), lambda i: (i, 0))
_deg_spec = pl.BlockSpec((_ROW_BLK, 1), lambda i: (i, 0))
_w_spec = pl.BlockSpec((H, H), lambda i: (0, 0))
_b_spec = pl.BlockSpec((1, H), lambda i: (0, 0))


def _tc_proj_body(xu, xi, wu, bu, wi, bi, deg_ru, deg_qi, hsu, hsi):
    hu = jnp.dot(xu[...], wu[...], preferred_element_type=jnp.float32) + bu[...]
    hsu[...] = hu * deg_ru[...]
    hi = jnp.dot(xi[...], wi[...], preferred_element_type=jnp.float32) + bi[...]
    hsi[...] = hi * deg_qi[...]


def _tc_proj(xu, xi, wu, bu, wi, bi, deg_ru, deg_qi):
    return pl.pallas_call(
        _tc_proj_body,
        grid=(_ROW_GRID,),
        in_specs=[_row_spec, _row_spec, _w_spec, _b_spec, _w_spec, _b_spec,
                  _deg_spec, _deg_spec],
        out_specs=[_row_spec] * 2,
        out_shape=[jax.ShapeDtypeStruct((NPAD, H), jnp.float32)] * 2,
    )(xu, xi, wu, bu.reshape(1, H), wi, bi.reshape(1, H), deg_ru, deg_qi)


def _tc_layer_body(relu, aggi, aggu, w_rates, b_rates, w_rated, b_rated,
                   deg_ri, deg_qu, deg_qi, deg_ru, hsu, hsi):
    hi = (jnp.dot(aggi[...], w_rates[...], preferred_element_type=jnp.float32)
          * deg_ri[...] + b_rates[...])
    hu = (jnp.dot(aggu[...], w_rated[...], preferred_element_type=jnp.float32)
          * deg_qu[...] + b_rated[...])
    if relu:
        hi = jnp.maximum(hi, 0.0)
        hu = jnp.maximum(hu, 0.0)
    hsi[...] = hi * deg_qi[...]
    hsu[...] = hu * deg_ru[...]


def _tc_layer(aggi, aggu, w_rates, b_rates, w_rated, b_rated,
              deg_ri, deg_qu, deg_qi, deg_ru, relu):
    return pl.pallas_call(
        functools.partial(_tc_layer_body, relu),
        grid=(_ROW_GRID,),
        in_specs=[_row_spec, _row_spec, _w_spec, _b_spec, _w_spec, _b_spec,
                  _deg_spec, _deg_spec, _deg_spec, _deg_spec],
        out_specs=[_row_spec] * 2,
        out_shape=[jax.ShapeDtypeStruct((NPAD, H), jnp.float32)] * 2,
    )(aggi, aggu, w_rates, b_rates.reshape(1, H), w_rated, b_rated.reshape(1, H),
      deg_ri, deg_qu, deg_qi, deg_ru)


def _tc_final_body(aggi, aggu, w_rates, b_rates, w_rated, b_rated,
                   deg_ri, deg_qu, ftw_u, ftw_i, ft_b, u_out, v_out):
    hi = (jnp.dot(aggi[...], w_rates[...], preferred_element_type=jnp.float32)
          * deg_ri[...] + b_rates[...])
    hu = (jnp.dot(aggu[...], w_rated[...], preferred_element_type=jnp.float32)
          * deg_qu[...] + b_rated[...])
    u_out[...] = jnp.dot(hu, ftw_u[...], preferred_element_type=jnp.float32)
    v_out[...] = jnp.dot(hi, ftw_i[...], preferred_element_type=jnp.float32) + ft_b[...]


def _tc_final(aggi, aggu, w_rates, b_rates, w_rated, b_rated,
              deg_ri, deg_qu, ftw_u, ftw_i, ft_b):
    return pl.pallas_call(
        _tc_final_body,
        grid=(_ROW_GRID,),
        in_specs=[_row_spec, _row_spec, _w_spec, _b_spec, _w_spec, _b_spec,
                  _deg_spec, _deg_spec, _w_spec, _w_spec, _b_spec],
        out_specs=[_row_spec] * 2,
        out_shape=[jax.ShapeDtypeStruct((NPAD, H), jnp.float32)] * 2,
    )(aggi, aggu, w_rates, b_rates.reshape(1, H), w_rated, b_rated.reshape(1, H),
      deg_ri, deg_qu, ftw_u, ftw_i, ft_b.reshape(1, H))


_SCORE_BLK = 3200
_SCORE_GRID = E // _SCORE_BLK


def _tc_score_body(gp, gn, bn_g, bn_b, bn_mean, bn_var, emb, w1, b1, w2, b2,
                   pos_out, neg_out):
    def score(g):
        t = jnp.maximum(g[...], 0.0)
        t = (t - bn_mean[...]) / jnp.sqrt(bn_var[...] + 1e-5) * bn_g[...] + bn_b[...]
        c = t + emb[...]
        z = jnp.maximum(jnp.dot(c, w1[...], preferred_element_type=jnp.float32)
                        + b1[...], 0.0)
        s = jnp.dot(z, w2[...], preferred_element_type=jnp.float32)[:, 0] + b2[0, 0]
        return s.reshape(_SCORE_BLK // CHUNK, CHUNK)

    i = pl.program_id(0)
    nrow = _SCORE_BLK // CHUNK
    pos_out[pl.ds(i * nrow, nrow), :] = score(gp)
    neg_out[pl.ds(i * nrow, nrow), :] = score(gn)


def _tc_score(gp, gn, bn_g, bn_b, bn_mean, bn_var, emb, w1, b1, w2, b2):
    full = lambda r, c: pl.BlockSpec((r, c), lambda i: (0, 0))
    return pl.pallas_call(
        _tc_score_body,
        grid=(_SCORE_GRID,),
        in_specs=[
            pl.BlockSpec((_SCORE_BLK, H), lambda i: (i, 0)),
            pl.BlockSpec((_SCORE_BLK, H), lambda i: (i, 0)),
            full(1, H), full(1, H), full(1, H), full(1, H), full(1, H),
            full(H, H // 2), full(1, H // 2), full(H // 2, 1), full(1, 1),
        ],
        out_specs=[pl.BlockSpec((NCHUNK, CHUNK), lambda i: (0, 0))] * 2,
        out_shape=[jax.ShapeDtypeStruct((NCHUNK, CHUNK), jnp.float32)] * 2,
    )(gp, gn, bn_g.reshape(1, H), bn_b.reshape(1, H), bn_mean.reshape(1, H),
      bn_var.reshape(1, H), emb.reshape(1, H), w1, b1.reshape(1, H // 2),
      w2, b2.reshape(1, 1))


def kernel(x_user, x_item, rates_src, rates_dst, rated_src, rated_dst,
           pos_src, pos_dst, neg_src, neg_dst,
           ip_user_W, ip_user_b, ip_item_W, ip_item_b,
           l0_rates_W, l0_rates_b, l0_rated_W, l0_rated_b,
           l1_rates_W, l1_rates_b, l1_rated_W, l1_rated_b,
           l2_rates_W, l2_rates_b, l2_rated_W, l2_rated_b,
           ft_W, ft_b, bn_g, bn_b, bn_mean, bn_var, emb,
           ep_W1, ep_b1, ep_W2, ep_b2):
    fill = N_NODE + (jnp.arange(E_PAD - E, dtype=jnp.int32) % (NPAD - N_NODE))
    c2 = lambda a: jnp.concatenate(
        [a.astype(jnp.int32), fill]).reshape(NCHUNK_PAD, CHUNK)
    rs, rd = c2(rates_src), c2(rates_dst)
    qs, qd = c2(rated_src), c2(rated_dst)
    ps, pd = c2(pos_src), c2(pos_dst)
    ns, nd = c2(neg_src), c2(neg_dst)
    zeros1 = jnp.zeros((DEG_PAD,), jnp.float32)
    zeros2 = jnp.zeros((NPAD, H), jnp.float32)

    deg_ru, deg_ri, deg_qi, deg_qu = _sc_degrees(rs, rd, qs, qd, zeros1)
    col = lambda a: (jnp.clip(a, 1.0, None) ** -0.5)[:NPAD].reshape(NPAD, 1)
    deg_ru, deg_ri, deg_qi, deg_qu = col(deg_ru), col(deg_ri), col(deg_qi), col(deg_qu)

    xpad = jnp.zeros((NPAD - N_NODE, H), jnp.float32)
    xu_p = jnp.concatenate([x_user, xpad])
    xi_p = jnp.concatenate([x_item, xpad])
    hsu, hsi = _tc_proj(xu_p, xi_p, ip_user_W, ip_user_b,
                        ip_item_W, ip_item_b, deg_ru, deg_qi)

    layer_w = [(l0_rates_W, l0_rates_b, l0_rated_W, l0_rated_b),
               (l1_rates_W, l1_rates_b, l1_rated_W, l1_rated_b)]
    for w_rates, b_rates, w_rated, b_rated in layer_w:
        aggi, aggu = _sc_aggregate(hsu, hsi, rs, rd, qs, qd, zeros2)
        hsu, hsi = _tc_layer(aggi, aggu, w_rates, b_rates, w_rated, b_rated,
                             deg_ri, deg_qu, deg_qi, deg_ru, relu=True)

    aggi, aggu = _sc_aggregate(hsu, hsi, rs, rd, qs, qd, zeros2)
    u_tbl, v_tbl = _tc_final(aggi, aggu, l2_rates_W, l2_rates_b,
                             l2_rated_W, l2_rated_b, deg_ri, deg_qu,
                             ft_W[:H], ft_W[H:], ft_b)

    g_pos, g_neg = _sc_edge_gather(u_tbl, v_tbl, ps, pd, ns, nd)
    pos2, neg2 = _tc_score(g_pos, g_neg, bn_g, bn_b, bn_mean, bn_var, emb,
                           ep_W1, ep_b1, ep_W2, ep_b2)
    return pos2.reshape(E), neg2.reshape(E)


# speedup vs baseline: 1.1689x; 1.1539x over previous
import functools

import jax
import jax.numpy as jnp
from jax import lax
from jax.experimental import pallas as pl
from jax.experimental.pallas import tpu as pltpu
from jax.experimental.pallas import tpu_sc as plsc

N_NODE = 10000
NPAD = 10112
E = 320000
H = 128
CHUNK = 128
NCHUNK = E // CHUNK
NSUB = 16
CPS = 162
NCHUNK_PAD = CPS * NSUB
E_PAD = NCHUNK_PAD * CHUNK
SEG = NPAD // NSUB
DEG_PAD = 10240
SEG_DEG = DEG_PAD // NSUB

_MESH = plsc.VectorSubcoreMesh(core_axis_name="c", subcore_axis_name="s")


_NB_DEG = 6


@functools.partial(
    pl.kernel,
    out_type=[jax.ShapeDtypeStruct((DEG_PAD,), jnp.float32)] * 4,
    mesh=_MESH,
    scratch_types=[
        pltpu.VMEM_SHARED((DEG_PAD,), jnp.float32),
        pltpu.VMEM_SHARED((DEG_PAD,), jnp.float32),
        [pltpu.VMEM((CHUNK,), jnp.int32)] * _NB_DEG,
        [pltpu.VMEM((CHUNK,), jnp.int32)] * _NB_DEG,
        pltpu.VMEM((CHUNK,), jnp.float32),
        [pltpu.SemaphoreType.DMA] * _NB_DEG,
        [pltpu.SemaphoreType.DMA] * _NB_DEG,
        [pltpu.SemaphoreType.DMA] * _NB_DEG,
        [pltpu.SemaphoreType.DMA] * _NB_DEG,
    ],
)
def _sc_degrees(rs, rd, qs, qd, zeros1, out0, out1, out2, out3,
                dega, degb, idxa, idxb, ones_v, ia, ib, sa, sb):
    cid = lax.axis_index("c")
    sid = lax.axis_index("s")
    base = sid * CPS
    for i in range(CHUNK // 16):
        ones_v[pl.ds(i * 16, 16)] = jnp.full((16,), 1.0, jnp.float32)
    pltpu.sync_copy(zeros1.at[pl.ds(sid * SEG_DEG, SEG_DEG)], dega.at[pl.ds(sid * SEG_DEG, SEG_DEG)])
    pltpu.sync_copy(zeros1.at[pl.ds(sid * SEG_DEG, SEG_DEG)], degb.at[pl.ds(sid * SEG_DEG, SEG_DEG)])
    plsc.subcore_barrier()

    def run(s2d, d2d):
        for b in range(_NB_DEG):
            pltpu.async_copy(s2d.at[base + b], idxa[b], ia[b])
            pltpu.async_copy(d2d.at[base + b], idxb[b], ib[b])

        def body(i, _):
            descs = []
            for b in range(_NB_DEG):
                j = i * _NB_DEG + b
                c = base + j
                pltpu.make_async_copy(s2d.at[c], idxa[b], ia[b]).wait()
                pltpu.make_async_copy(d2d.at[c], idxb[b], ib[b]).wait()
                descs.append(
                    (pltpu.async_copy(ones_v, dega.at[idxa[b]], sa[b], add=True),
                     pltpu.async_copy(ones_v, degb.at[idxb[b]], sb[b], add=True)))
            for b in range(_NB_DEG):
                j = i * _NB_DEG + b
                c = base + j
                descs[b][0].wait()
                descs[b][1].wait()

                @pl.when(j + _NB_DEG < CPS)
                def _():
                    pltpu.async_copy(s2d.at[c + _NB_DEG], idxa[b], ia[b])
                    pltpu.async_copy(d2d.at[c + _NB_DEG], idxb[b], ib[b])
            return 0
        lax.fori_loop(0, CPS // _NB_DEG, body, 0)

    @pl.when(cid == 0)
    def _():
        run(rs, rd)

    @pl.when(cid == 1)
    def _():
        run(qs, qd)

    plsc.subcore_barrier()

    @pl.when(cid == 0)
    def _():
        pltpu.sync_copy(dega.at[pl.ds(sid * SEG_DEG, SEG_DEG)], out0.at[pl.ds(sid * SEG_DEG, SEG_DEG)])
        pltpu.sync_copy(degb.at[pl.ds(sid * SEG_DEG, SEG_DEG)], out1.at[pl.ds(sid * SEG_DEG, SEG_DEG)])

    @pl.when(cid == 1)
    def _():
        pltpu.sync_copy(dega.at[pl.ds(sid * SEG_DEG, SEG_DEG)], out2.at[pl.ds(sid * SEG_DEG, SEG_DEG)])
        pltpu.sync_copy(degb.at[pl.ds(sid * SEG_DEG, SEG_DEG)], out3.at[pl.ds(sid * SEG_DEG, SEG_DEG)])


_NB_AGG = 3


@functools.partial(
    pl.kernel,
    out_type=[jax.ShapeDtypeStruct((NPAD, H), jnp.float32)] * 2,
    mesh=_MESH,
    scratch_types=[
        pltpu.VMEM_SHARED((NPAD, H), jnp.float32),
        [pltpu.VMEM((CHUNK,), jnp.int32)] * _NB_AGG,
        [pltpu.VMEM((CHUNK,), jnp.int32)] * _NB_AGG,
        [pltpu.VMEM((CHUNK, H), jnp.float32)] * _NB_AGG,
        [pltpu.SemaphoreType.DMA] * _NB_AGG,
        [pltpu.SemaphoreType.DMA] * _NB_AGG,
        [pltpu.SemaphoreType.DMA] * _NB_AGG,
        [pltpu.SemaphoreType.DMA] * _NB_AGG,
    ],
)
def _sc_aggregate(hs_user, hs_item, rs, rd, qs, qd, zeros2,
                  agg_item, agg_user, accum, idxs, idxd, rows, ia, ib, gs, ss):
    cid = lax.axis_index("c")
    sid = lax.axis_index("s")
    base = sid * CPS
    pltpu.sync_copy(zeros2.at[pl.ds(sid * SEG, SEG)],
                    accum.at[pl.ds(sid * SEG, SEG)])
    plsc.subcore_barrier()

    def run(tbl, s2d, d2d):
        for b in range(_NB_AGG):
            pltpu.async_copy(s2d.at[base + b], idxs[b], ia[b])
            pltpu.async_copy(d2d.at[base + b], idxd[b], ib[b])
        for b in range(_NB_AGG):
            pltpu.make_async_copy(s2d.at[base + b], idxs[b], ia[b]).wait()
            pltpu.async_copy(tbl.at[idxs[b]], rows[b], gs[b])

        def body(i, _):
            sdescs = []
            for b in range(_NB_AGG):
                j = i * _NB_AGG + b
                c = base + j
                pltpu.make_async_copy(tbl.at[idxs[b]], rows[b], gs[b]).wait()

                @pl.when(j + _NB_AGG < CPS)
                def _():
                    pltpu.async_copy(s2d.at[c + _NB_AGG], idxs[b], ia[b])
                pltpu.make_async_copy(d2d.at[c], idxd[b], ib[b]).wait()
                sdescs.append(pltpu.async_copy(rows[b], accum.at[idxd[b]],
                                               ss[b], add=True))
            for b in range(_NB_AGG):
                j = i * _NB_AGG + b
                c = base + j
                sdescs[b].wait()

                @pl.when(j + _NB_AGG < CPS)
                def _():
                    pltpu.async_copy(d2d.at[c + _NB_AGG], idxd[b], ib[b])
                    pltpu.make_async_copy(s2d.at[c + _NB_AGG], idxs[b], ia[b]).wait()
                    pltpu.async_copy(tbl.at[idxs[b]], rows[b], gs[b])
            return 0
        lax.fori_loop(0, CPS // _NB_AGG, body, 0)

    @pl.when(cid == 0)
    def _():
        run(hs_user, rs, rd)

    @pl.when(cid == 1)
    def _():
        run(hs_item, qs, qd)

    plsc.subcore_barrier()

    @pl.when(cid == 0)
    def _():
        pltpu.sync_copy(accum.at[pl.ds(sid * SEG, SEG)],
                        agg_item.at[pl.ds(sid * SEG, SEG)])

    @pl.when(cid == 1)
    def _():
        pltpu.sync_copy(accum.at[pl.ds(sid * SEG, SEG)],
                        agg_user.at[pl.ds(sid * SEG, SEG)])


_NB_EDGE = 3


@functools.partial(
    pl.kernel,
    out_type=[jax.ShapeDtypeStruct((E_PAD, H), jnp.float32)] * 2,
    mesh=_MESH,
    scratch_types=[
        pltpu.VMEM_SHARED((NPAD, H), jnp.float32),
        [pltpu.VMEM((CHUNK,), jnp.int32)] * _NB_EDGE,
        [pltpu.VMEM((CHUNK,), jnp.int32)] * _NB_EDGE,
        [pltpu.VMEM((CHUNK, H), jnp.float32)] * _NB_EDGE,
        [pltpu.SemaphoreType.DMA] * _NB_EDGE,
        [pltpu.SemaphoreType.DMA] * _NB_EDGE,
        [pltpu.SemaphoreType.DMA] * _NB_EDGE,
        [pltpu.SemaphoreType.DMA] * _NB_EDGE,
        [pltpu.SemaphoreType.DMA] * _NB_EDGE,
    ],
)
def _sc_edge_gather(u_tbl, v_tbl, ps, pd, ns, nd, g_pos, g_neg,
                    utab, idxs, idxd, rows, ia, ib, us, vs, os):
    cid = lax.axis_index("c")
    sid = lax.axis_index("s")
    base = sid * CPS
    pltpu.sync_copy(u_tbl.at[pl.ds(sid * SEG, SEG)],
                    utab.at[pl.ds(sid * SEG, SEG)])
    plsc.subcore_barrier()

    def run(s2d, d2d, out):
        for b in range(_NB_EDGE):
            pltpu.async_copy(s2d.at[base + b], idxs[b], ia[b])
            pltpu.async_copy(d2d.at[base + b], idxd[b], ib[b])
        for b in range(_NB_EDGE):
            pltpu.make_async_copy(s2d.at[base + b], idxs[b], ia[b]).wait()
            pltpu.async_copy(utab.at[idxs[b]], rows[b], us[b])

        def body(i, _):
            vdescs = []
            for b in range(_NB_EDGE):
                j = i * _NB_EDGE + b
                c = base + j
                pltpu.make_async_copy(utab.at[idxs[b]], rows[b], us[b]).wait()

                @pl.when(j + _NB_EDGE < CPS)
                def _():
                    pltpu.async_copy(s2d.at[c + _NB_EDGE], idxs[b], ia[b])
                pltpu.make_async_copy(d2d.at[c], idxd[b], ib[b]).wait()
                vdescs.append(pltpu.async_copy(v_tbl.at[idxd[b]], rows[b],
                                               vs[b], add=True))
            odescs = []
            for b in range(_NB_EDGE):
                c = base + i * _NB_EDGE + b
                vdescs[b].wait()
                odescs.append(pltpu.async_copy(
                    rows[b], out.at[pl.ds(c * CHUNK, CHUNK)], os[b]))
            for b in range(_NB_EDGE):
                j = i * _NB_EDGE + b
                c = base + j
                odescs[b].wait()

                @pl.when(j + _NB_EDGE < CPS)
                def _():
                    pltpu.async_copy(d2d.at[c + _NB_EDGE], idxd[b], ib[b])
                    pltpu.make_async_copy(s2d.at[c + _NB_EDGE], idxs[b], ia[b]).wait()
                    pltpu.async_copy(utab.at[idxs[b]], rows[b], us[b])
            return 0
        lax.fori_loop(0, CPS // _NB_EDGE, body, 0)

    @pl.when(cid == 0)
    def _():
        run(ps, pd, g_pos)

    @pl.when(cid == 1)
    def _():
        run(ns, nd, g_neg)


_ROW_BLK = 1264
_ROW_GRID = NPAD // _ROW_BLK
_row_spec = pl.BlockSpec((_ROW_BLK, H), lambda i: (i, 0))
_deg_spec = pl.BlockSpec((_ROW_BLK, 1), lambda i: (i, 0))
_w_spec = pl.BlockSpec((H, H), lambda i: (0, 0))
_b_spec = pl.BlockSpec((1, H), lambda i: (0, 0))


def _tc_proj_body(xu, xi, wu, bu, wi, bi, deg_ru, deg_qi, hsu, hsi):
    hu = jnp.dot(xu[...], wu[...], preferred_element_type=jnp.float32) + bu[...]
    hsu[...] = hu * deg_ru[...]
    hi = jnp.dot(xi[...], wi[...], preferred_element_type=jnp.float32) + bi[...]
    hsi[...] = hi * deg_qi[...]


def _tc_proj(xu, xi, wu, bu, wi, bi, deg_ru, deg_qi):
    return pl.pallas_call(
        _tc_proj_body,
        grid=(_ROW_GRID,),
        in_specs=[_row_spec, _row_spec, _w_spec, _b_spec, _w_spec, _b_spec,
                  _deg_spec, _deg_spec],
        out_specs=[_row_spec] * 2,
        out_shape=[jax.ShapeDtypeStruct((NPAD, H), jnp.float32)] * 2,
    )(xu, xi, wu, bu.reshape(1, H), wi, bi.reshape(1, H), deg_ru, deg_qi)


def _tc_layer_body(relu, aggi, aggu, w_rates, b_rates, w_rated, b_rated,
                   deg_ri, deg_qu, deg_qi, deg_ru, hsu, hsi):
    hi = (jnp.dot(aggi[...], w_rates[...], preferred_element_type=jnp.float32)
          * deg_ri[...] + b_rates[...])
    hu = (jnp.dot(aggu[...], w_rated[...], preferred_element_type=jnp.float32)
          * deg_qu[...] + b_rated[...])
    if relu:
        hi = jnp.maximum(hi, 0.0)
        hu = jnp.maximum(hu, 0.0)
    hsi[...] = hi * deg_qi[...]
    hsu[...] = hu * deg_ru[...]


def _tc_layer(aggi, aggu, w_rates, b_rates, w_rated, b_rated,
              deg_ri, deg_qu, deg_qi, deg_ru, relu):
    return pl.pallas_call(
        functools.partial(_tc_layer_body, relu),
        grid=(_ROW_GRID,),
        in_specs=[_row_spec, _row_spec, _w_spec, _b_spec, _w_spec, _b_spec,
                  _deg_spec, _deg_spec, _deg_spec, _deg_spec],
        out_specs=[_row_spec] * 2,
        out_shape=[jax.ShapeDtypeStruct((NPAD, H), jnp.float32)] * 2,
    )(aggi, aggu, w_rates, b_rates.reshape(1, H), w_rated, b_rated.reshape(1, H),
      deg_ri, deg_qu, deg_qi, deg_ru)


def _tc_final_body(aggi, aggu, w_rates, b_rates, w_rated, b_rated,
                   deg_ri, deg_qu, ftw_u, ftw_i, ft_b, u_out, v_out):
    hi = (jnp.dot(aggi[...], w_rates[...], preferred_element_type=jnp.float32)
          * deg_ri[...] + b_rates[...])
    hu = (jnp.dot(aggu[...], w_rated[...], preferred_element_type=jnp.float32)
          * deg_qu[...] + b_rated[...])
    u_out[...] = jnp.dot(hu, ftw_u[...], preferred_element_type=jnp.float32)
    v_out[...] = jnp.dot(hi, ftw_i[...], preferred_element_type=jnp.float32) + ft_b[...]


def _tc_final(aggi, aggu, w_rates, b_rates, w_rated, b_rated,
              deg_ri, deg_qu, ftw_u, ftw_i, ft_b):
    return pl.pallas_call(
        _tc_final_body,
        grid=(_ROW_GRID,),
        in_specs=[_row_spec, _row_spec, _w_spec, _b_spec, _w_spec, _b_spec,
                  _deg_spec, _deg_spec, _w_spec, _w_spec, _b_spec],
        out_specs=[_row_spec] * 2,
        out_shape=[jax.ShapeDtypeStruct((NPAD, H), jnp.float32)] * 2,
    )(aggi, aggu, w_rates, b_rates.reshape(1, H), w_rated, b_rated.reshape(1, H),
      deg_ri, deg_qu, ftw_u, ftw_i, ft_b.reshape(1, H))


_SCORE_BLK = 3200
_SCORE_GRID = E // _SCORE_BLK


def _tc_score_body(gp, gn, bn_g, bn_b, bn_mean, bn_var, emb, w1, b1, w2, b2,
                   pos_out, neg_out):
    def score(g):
        t = jnp.maximum(g[...], 0.0)
        t = (t - bn_mean[...]) / jnp.sqrt(bn_var[...] + 1e-5) * bn_g[...] + bn_b[...]
        c = t + emb[...]
        z = jnp.maximum(jnp.dot(c, w1[...], preferred_element_type=jnp.float32)
                        + b1[...], 0.0)
        s = jnp.dot(z, w2[...], preferred_element_type=jnp.float32)[:, 0] + b2[0, 0]
        return s.reshape(_SCORE_BLK // CHUNK, CHUNK)

    i = pl.program_id(0)
    nrow = _SCORE_BLK // CHUNK
    pos_out[pl.ds(i * nrow, nrow), :] = score(gp)
    neg_out[pl.ds(i * nrow, nrow), :] = score(gn)


def _tc_score(gp, gn, bn_g, bn_b, bn_mean, bn_var, emb, w1, b1, w2, b2):
    full = lambda r, c: pl.BlockSpec((r, c), lambda i: (0, 0))
    return pl.pallas_call(
        _tc_score_body,
        grid=(_SCORE_GRID,),
        in_specs=[
            pl.BlockSpec((_SCORE_BLK, H), lambda i: (i, 0)),
            pl.BlockSpec((_SCORE_BLK, H), lambda i: (i, 0)),
            full(1, H), full(1, H), full(1, H), full(1, H), full(1, H),
            full(H, H // 2), full(1, H // 2), full(H // 2, 1), full(1, 1),
        ],
        out_specs=[pl.BlockSpec((NCHUNK, CHUNK), lambda i: (0, 0))] * 2,
        out_shape=[jax.ShapeDtypeStruct((NCHUNK, CHUNK), jnp.float32)] * 2,
    )(gp, gn, bn_g.reshape(1, H), bn_b.reshape(1, H), bn_mean.reshape(1, H),
      bn_var.reshape(1, H), emb.reshape(1, H), w1, b1.reshape(1, H // 2),
      w2, b2.reshape(1, 1))


def kernel(x_user, x_item, rates_src, rates_dst, rated_src, rated_dst,
           pos_src, pos_dst, neg_src, neg_dst,
           ip_user_W, ip_user_b, ip_item_W, ip_item_b,
           l0_rates_W, l0_rates_b, l0_rated_W, l0_rated_b,
           l1_rates_W, l1_rates_b, l1_rated_W, l1_rated_b,
           l2_rates_W, l2_rates_b, l2_rated_W, l2_rated_b,
           ft_W, ft_b, bn_g, bn_b, bn_mean, bn_var, emb,
           ep_W1, ep_b1, ep_W2, ep_b2):
    fill = N_NODE + (jnp.arange(E_PAD - E, dtype=jnp.int32) % (NPAD - N_NODE))
    c2 = lambda a: jnp.concatenate(
        [a.astype(jnp.int32), fill]).reshape(NCHUNK_PAD, CHUNK)
    rs, rd = c2(rates_src), c2(rates_dst)
    qs, qd = c2(rated_src), c2(rated_dst)
    ps, pd = c2(pos_src), c2(pos_dst)
    ns, nd = c2(neg_src), c2(neg_dst)
    zeros1 = jnp.zeros((DEG_PAD,), jnp.float32)
    zeros2 = jnp.zeros((NPAD, H), jnp.float32)

    deg_ru, deg_ri, deg_qi, deg_qu = _sc_degrees(rs, rd, qs, qd, zeros1)
    col = lambda a: (jnp.clip(a, 1.0, None) ** -0.5)[:NPAD].reshape(NPAD, 1)
    deg_ru, deg_ri, deg_qi, deg_qu = col(deg_ru), col(deg_ri), col(deg_qi), col(deg_qu)

    xpad = jnp.zeros((NPAD - N_NODE, H), jnp.float32)
    xu_p = jnp.concatenate([x_user, xpad])
    xi_p = jnp.concatenate([x_item, xpad])
    hsu, hsi = _tc_proj(xu_p, xi_p, ip_user_W, ip_user_b,
                        ip_item_W, ip_item_b, deg_ru, deg_qi)

    layer_w = [(l0_rates_W, l0_rates_b, l0_rated_W, l0_rated_b),
               (l1_rates_W, l1_rates_b, l1_rated_W, l1_rated_b)]
    for w_rates, b_rates, w_rated, b_rated in layer_w:
        aggi, aggu = _sc_aggregate(hsu, hsi, rs, rd, qs, qd, zeros2)
        hsu, hsi = _tc_layer(aggi, aggu, w_rates, b_rates, w_rated, b_rated,
                             deg_ri, deg_qu, deg_qi, deg_ru, relu=True)

    aggi, aggu = _sc_aggregate(hsu, hsi, rs, rd, qs, qd, zeros2)
    u_tbl, v_tbl = _tc_final(aggi, aggu, l2_rates_W, l2_rates_b,
                             l2_rated_W, l2_rated_b, deg_ri, deg_qu,
                             ft_W[:H], ft_W[H:], ft_b)

    g_pos, g_neg = _sc_edge_gather(u_tbl, v_tbl, ps, pd, ns, nd)
    pos2, neg2 = _tc_score(g_pos, g_neg, bn_g, bn_b, bn_mean, bn_var, emb,
                           ep_W1, ep_b1, ep_W2, ep_b2)
    return pos2.reshape(E), neg2.reshape(E)
